# Initial kernel scaffold; baseline (speedup 1.0000x reference)
import jax
import jax.numpy as jnp
from jax.experimental import pallas as pl


def kernel(x, table):
    raise NotImplementedError("write your pallas kernel here")


# speedup vs baseline: 1.0984x; 1.0984x over previous
import functools

import jax
import jax.numpy as jnp
from jax import lax
from jax.experimental import pallas as pl
from jax.experimental.pallas import tpu as pltpu
from jax.experimental.pallas import tpu_sc as plsc

D = 32
NW = 32
G = 128
K = 8
CH = K * G


def _make_sc_gather(B):
    per_w = B // NW
    ng = per_w // G
    ngrp = ng // K
    mesh = plsc.VectorSubcoreMesh(core_axis_name="c", subcore_axis_name="s")

    @functools.partial(
        pl.kernel,
        mesh=mesh,
        out_type=jax.ShapeDtypeStruct((B, D), jnp.float32),
        scratch_types=[
            pltpu.VMEM((ng, G), jnp.int32),
            pltpu.VMEM((CH, D), jnp.float32),
            pltpu.SemaphoreType.DMA,
        ],
        compiler_params=pltpu.CompilerParams(use_tc_tiling_on_sc=False),
    )
    def sc_gather(x_hbm, table_hbm, out_hbm, idx_v, rows_v, sem):
        wid = lax.axis_index("s") * 2 + lax.axis_index("c")
        pltpu.sync_copy(x_hbm.at[wid], idx_v)
        out_base = wid * per_w

        def body(g, carry):
            cps = [
                pltpu.async_copy(
                    table_hbm.at[idx_v.at[g * K + jj]],
                    rows_v.at[pl.ds(jj * G, G)],
                    sem,
                )
                for jj in range(K)
            ]
            for cp in cps:
                cp.wait()
            pltpu.sync_copy(rows_v, out_hbm.at[pl.ds(out_base + g * CH, CH)])
            return carry

        lax.fori_loop(0, ngrp, body, 0)

    return sc_gather


def _mask_body(x_ref, o_ref):
    o_ref[...] = x_ref[...] > 0


def kernel(x, table):
    bsz, hist = x.shape
    B = bsz * hist
    idx3 = x.reshape(NW, (B // NW) // G, G)
    embs = _make_sc_gather(B)(idx3, table).reshape(bsz, hist, D)

    rows_blk = bsz // 16
    mask = pl.pallas_call(
        _mask_body,
        grid=(16,),
        in_specs=[pl.BlockSpec((rows_blk, hist), lambda i: (i, 0))],
        out_specs=pl.BlockSpec((rows_blk, hist), lambda i: (i, 0)),
        out_shape=jax.ShapeDtypeStruct((bsz, hist), jnp.bool_),
    )(x)
    return (embs, mask)


# speedup vs baseline: 1.4839x; 1.3509x over previous
import functools

import jax
import jax.numpy as jnp
from jax import lax
from jax.experimental import pallas as pl
from jax.experimental.pallas import tpu as pltpu
from jax.experimental.pallas import tpu_sc as plsc

D = 32
NW = 32
G = 128


def _make_sc_gather(bsz, hist, V):
    bw = bsz // NW
    nlb = bw // G
    nb_all = bsz // G
    mesh = plsc.VectorSubcoreMesh(core_axis_name="c", subcore_axis_name="s")

    @functools.partial(
        pl.kernel,
        mesh=mesh,
        out_type=jax.ShapeDtypeStruct((hist, 4, nb_all, 8, G), jnp.float32),
        scratch_types=[
            pltpu.VMEM((bw, hist), jnp.int32),
            pltpu.VMEM((hist, bw), jnp.int32),
            pltpu.VMEM((bw, D), jnp.float32),
            pltpu.VMEM((4, nlb, 8, G), jnp.float32),
            pltpu.SemaphoreType.DMA,
        ],
        compiler_params=pltpu.CompilerParams(
            use_tc_tiling_on_sc=False, needs_layout_passes=False
        ),
    )
    def sc_gather(x_hbm, table_hbm, out_hbm, idx_raw, idx_t, gbuf, colbuf, sem):
        wid = lax.axis_index("s") * 2 + lax.axis_index("c")
        pltpu.sync_copy(x_hbm.at[wid], idx_raw)
        nb0 = wid * nlb
        lane = lax.iota(jnp.int32, 16)

        def idx_tr_body(h, carry):
            cols = jnp.full((16,), h, jnp.int32)
            for q in range(bw // 16):
                v = plsc.load_gather(idx_raw, [lane + q * 16, cols])
                idx_t[h, pl.ds(q * 16, 16)] = v
            return carry

        lax.fori_loop(0, hist, idx_tr_body, 0)

        def emb_tr_body(r, carry):
            roff = r * 16
            for lb in range(nlb):
                rows = lane + lb * G + roff
                for j in range(D):
                    cols = jnp.full((16,), j, jnp.int32)
                    v = plsc.load_gather(gbuf, [rows, cols])
                    colbuf[j // 8, lb, j % 8, pl.ds(roff, 16)] = v
            return carry

        def body(h, carry):
            cps = [
                pltpu.async_copy(
                    table_hbm.at[idx_t.at[h, pl.ds(lb * G, G)]],
                    gbuf.at[pl.ds(lb * G, G)],
                    sem,
                )
                for lb in range(nlb)
            ]
            for cp in cps:
                cp.wait()
            lax.fori_loop(0, G // 16, emb_tr_body, 0)
            for tj in range(4):
                pltpu.sync_copy(
                    colbuf.at[tj], out_hbm.at[h, tj, pl.ds(nb0, nlb)]
                )
            return carry

        lax.fori_loop(0, hist, body, 0)

    return sc_gather


def _mask_body(x_ref, o_ref):
    o_ref[...] = x_ref[...] > 0


def kernel(x, table):
    bsz, hist = x.shape
    V, _ = table.shape
    x3 = x.reshape(NW, bsz // NW, hist)
    o6 = _make_sc_gather(bsz, hist, V)(x3, table)
    embs = (
        o6.transpose(2, 4, 0, 1, 3)
        .reshape(bsz, hist, D)
    )

    rows_blk = bsz // 16
    mask = pl.pallas_call(
        _mask_body,
        grid=(16,),
        in_specs=[pl.BlockSpec((rows_blk, hist), lambda i: (i, 0))],
        out_specs=pl.BlockSpec((rows_blk, hist), lambda i: (i, 0)),
        out_shape=jax.ShapeDtypeStruct((bsz, hist), jnp.bool_),
    )(x)
    return (embs, mask)


# speedup vs baseline: 2.2498x; 1.5162x over previous
import functools

import jax
import jax.numpy as jnp
from jax import lax
from jax.experimental import pallas as pl
from jax.experimental.pallas import tpu as pltpu
from jax.experimental.pallas import tpu_sc as plsc

D = 32
NW = 32
G = 128


def _make_sc_gather(bsz, hist, V):
    bw = bsz // NW
    nlb = bw // G
    nb_all = bsz // G
    mesh = plsc.VectorSubcoreMesh(core_axis_name="c", subcore_axis_name="s")

    @functools.partial(
        pl.kernel,
        mesh=mesh,
        out_type=jax.ShapeDtypeStruct((hist, 4, nb_all, 8, G), jnp.float32),
        scratch_types=[
            pltpu.VMEM((bw, hist), jnp.int32),
            pltpu.VMEM((hist, bw), jnp.int32),
            pltpu.VMEM((bw, D), jnp.float32),
            pltpu.VMEM((4, nlb, 8, G), jnp.float32),
            pltpu.SemaphoreType.DMA,
        ],
        compiler_params=pltpu.CompilerParams(
            use_tc_tiling_on_sc=False, needs_layout_passes=False
        ),
    )
    def sc_gather(x_hbm, table_hbm, out_hbm, idx_raw, idx_t, gbuf, colbuf, sem):
        wid = lax.axis_index("s") * 2 + lax.axis_index("c")
        pltpu.sync_copy(x_hbm.at[wid], idx_raw)
        nb0 = wid * nlb
        lane = lax.iota(jnp.int32, 16)

        def idx_tr_body(h, carry):
            cols = jnp.full((16,), h, jnp.int32)
            for q in range(bw // 16):
                v = plsc.load_gather(idx_raw, [lane + q * 16, cols])
                idx_t[h, pl.ds(q * 16, 16)] = v
            return carry

        lax.fori_loop(0, hist, idx_tr_body, 0)

        def emb_tr_body(d, carry):
            jm = (d + lane) & (D - 1)
            tj_v = jm >> 3
            dr_v = jm & 7
            for lb in range(nlb):
                lb_v = jnp.full((16,), lb, jnp.int32)
                for r in range(G // 16):
                    rows = lane + (lb * G + r * 16)
                    v = plsc.load_gather(gbuf, [rows, jm])
                    plsc.store_scatter(
                        colbuf, [tj_v, lb_v, dr_v, lane + r * 16], v
                    )
            return carry

        def body(h, carry):
            cps = [
                pltpu.async_copy(
                    table_hbm.at[idx_t.at[h, pl.ds(lb * G, G)]],
                    gbuf.at[pl.ds(lb * G, G)],
                    sem,
                )
                for lb in range(nlb)
            ]
            for cp in cps:
                cp.wait()
            lax.fori_loop(0, D, emb_tr_body, 0)
            for tj in range(4):
                pltpu.sync_copy(
                    colbuf.at[tj], out_hbm.at[h, tj, pl.ds(nb0, nlb)]
                )
            return carry

        lax.fori_loop(0, hist, body, 0)

    return sc_gather


def _mask_body(x_ref, o_ref):
    o_ref[...] = x_ref[...] > 0


def kernel(x, table):
    bsz, hist = x.shape
    V, _ = table.shape
    x3 = x.reshape(NW, bsz // NW, hist)
    o6 = _make_sc_gather(bsz, hist, V)(x3, table)
    embs = (
        o6.transpose(2, 4, 0, 1, 3)
        .reshape(bsz, hist, D)
    )

    rows_blk = bsz // 16
    mask = pl.pallas_call(
        _mask_body,
        grid=(16,),
        in_specs=[pl.BlockSpec((rows_blk, hist), lambda i: (i, 0))],
        out_specs=pl.BlockSpec((rows_blk, hist), lambda i: (i, 0)),
        out_shape=jax.ShapeDtypeStruct((bsz, hist), jnp.bool_),
    )(x)
    return (embs, mask)


# speedup vs baseline: 2.2827x; 1.0147x over previous
import functools

import jax
import jax.numpy as jnp
from jax import lax
from jax.experimental import pallas as pl
from jax.experimental.pallas import tpu as pltpu
from jax.experimental.pallas import tpu_sc as plsc

D = 32
NW = 32
G = 128


def _make_sc_gather(bsz, hist, V):
    bw = bsz // NW
    nlb = bw // G
    nb_all = bsz // G
    mesh = plsc.VectorSubcoreMesh(core_axis_name="c", subcore_axis_name="s")

    @functools.partial(
        pl.kernel,
        mesh=mesh,
        out_type=jax.ShapeDtypeStruct((hist, 4, nb_all, 8, G), jnp.float32),
        scratch_types=[
            pltpu.VMEM((bw, hist), jnp.int32),
            pltpu.VMEM((hist, bw), jnp.int32),
            pltpu.VMEM((2, bw, D), jnp.float32),
            pltpu.VMEM((4, nlb, 8, G), jnp.float32),
            pltpu.SemaphoreType.DMA,
            pltpu.SemaphoreType.DMA,
        ],
        compiler_params=pltpu.CompilerParams(
            use_tc_tiling_on_sc=False, needs_layout_passes=False
        ),
    )
    def sc_gather(x_hbm, table_hbm, out_hbm, idx_raw, idx_t, gbuf, colbuf, sem_a, sem_b):
        wid = lax.axis_index("s") * 2 + lax.axis_index("c")
        pltpu.sync_copy(x_hbm.at[wid], idx_raw)
        nb0 = wid * nlb
        lane = lax.iota(jnp.int32, 16)

        def idx_tr_body(h, carry):
            cols = jnp.full((16,), h, jnp.int32)
            for q in range(bw // 16):
                v = plsc.load_gather(idx_raw, [lane + q * 16, cols])
                idx_t[h, pl.ds(q * 16, 16)] = v
            return carry

        lax.fori_loop(0, hist, idx_tr_body, 0)

        def fire(h, buf, sem):
            return [
                pltpu.async_copy(
                    table_hbm.at[idx_t.at[h, pl.ds(lb * G, G)]],
                    gbuf.at[buf, pl.ds(lb * G, G)],
                    sem,
                )
                for lb in range(nlb)
            ]

        def transpose_chunk(buf):
            def emb_tr_body(d, carry):
                jm = (d + lane) & (D - 1)
                tj_v = jm >> 3
                dr_v = jm & 7
                for lb in range(nlb):
                    lb_v = jnp.full((16,), lb, jnp.int32)
                    for r in range(G // 16):
                        rows = lane + (lb * G + r * 16)
                        v = plsc.load_gather(gbuf.at[buf], [rows, jm])
                        plsc.store_scatter(
                            colbuf, [tj_v, lb_v, dr_v, lane + r * 16], v
                        )
                return carry

            lax.fori_loop(0, D, emb_tr_body, 0)

        def write_out(h):
            for tj in range(4):
                pltpu.sync_copy(
                    colbuf.at[tj], out_hbm.at[h, tj, pl.ds(nb0, nlb)]
                )

        def body(hh, carry):
            h0 = hh * 2
            h1 = h0 + 1
            cps_a = fire(h0, 0, sem_a)
            cps_b = fire(h1, 1, sem_b)
            for cp in cps_a:
                cp.wait()
            transpose_chunk(0)
            write_out(h0)
            for cp in cps_b:
                cp.wait()
            transpose_chunk(1)
            write_out(h1)
            return carry

        lax.fori_loop(0, hist // 2, body, 0)

    return sc_gather


def _mask_body(x_ref, o_ref):
    o_ref[...] = x_ref[...] > 0


def kernel(x, table):
    bsz, hist = x.shape
    V, _ = table.shape
    x3 = x.reshape(NW, bsz // NW, hist)
    o6 = _make_sc_gather(bsz, hist, V)(x3, table)
    embs = (
        o6.transpose(2, 4, 0, 1, 3)
        .reshape(bsz, hist, D)
    )

    rows_blk = bsz // 16
    mask = pl.pallas_call(
        _mask_body,
        grid=(16,),
        in_specs=[pl.BlockSpec((rows_blk, hist), lambda i: (i, 0))],
        out_specs=pl.BlockSpec((rows_blk, hist), lambda i: (i, 0)),
        out_shape=jax.ShapeDtypeStruct((bsz, hist), jnp.bool_),
    )(x)
    return (embs, mask)


# speedup vs baseline: 2.3904x; 1.0472x over previous
import functools

import jax
import jax.numpy as jnp
from jax import lax
from jax.experimental import pallas as pl
from jax.experimental.pallas import tpu as pltpu
from jax.experimental.pallas import tpu_sc as plsc

D = 32
NW = 32
G = 128


def _make_sc_gather(bsz, hist, V):
    bw = bsz // NW
    nlb = bw // G
    nb_all = bsz // G
    mesh = plsc.VectorSubcoreMesh(core_axis_name="c", subcore_axis_name="s")

    @functools.partial(
        pl.kernel,
        mesh=mesh,
        out_type=jax.ShapeDtypeStruct((hist, 4, nb_all, 8, G), jnp.float32),
        scratch_types=[
            pltpu.VMEM((bw, hist), jnp.int32),
            pltpu.VMEM((hist, bw), jnp.int32),
            pltpu.VMEM((2, bw, D), jnp.float32),
            pltpu.VMEM((4, nlb, 8, G), jnp.float32),
            pltpu.SemaphoreType.DMA,
            pltpu.SemaphoreType.DMA,
        ],
        compiler_params=pltpu.CompilerParams(
            use_tc_tiling_on_sc=False, needs_layout_passes=False
        ),
    )
    def sc_gather(x_hbm, table_hbm, out_hbm, idx_raw, idx_t, gbuf, colbuf, sem_a, sem_b):
        wid = lax.axis_index("s") * 2 + lax.axis_index("c")
        pltpu.sync_copy(x_hbm.at[wid], idx_raw)
        nb0 = wid * nlb
        lane = lax.iota(jnp.int32, 16)

        def idx_tr_body(h, carry):
            cols = jnp.full((16,), h, jnp.int32)
            for q in range(bw // 16):
                v = plsc.load_gather(idx_raw, [lane + q * 16, cols])
                idx_t[h, pl.ds(q * 16, 16)] = v
            return carry

        lax.fori_loop(0, hist, idx_tr_body, 0)

        def fire(h, buf, sem):
            for lb in range(nlb):
                pltpu.async_copy(
                    table_hbm.at[idx_t.at[h, pl.ds(lb * G, G)]],
                    gbuf.at[buf, pl.ds(lb * G, G)],
                    sem,
                )

        def drain(buf, sem):
            for lb in range(nlb):
                pltpu.make_async_copy(
                    table_hbm.at[pl.ds(0, G)],
                    gbuf.at[buf, pl.ds(lb * G, G)],
                    sem,
                ).wait()

        def transpose_chunk(buf):
            def emb_tr_body(d, carry):
                jm = (d + lane) & (D - 1)
                tj_v = jm >> 3
                dr_v = jm & 7
                for lb in range(nlb):
                    lb_v = jnp.full((16,), lb, jnp.int32)
                    for r in range(G // 16):
                        rows = lane + (lb * G + r * 16)
                        v = plsc.load_gather(gbuf.at[buf], [rows, jm])
                        plsc.store_scatter(
                            colbuf, [tj_v, lb_v, dr_v, lane + r * 16], v
                        )
                return carry

            lax.fori_loop(0, D, emb_tr_body, 0)

        def write_out(h):
            for tj in range(4):
                pltpu.sync_copy(
                    colbuf.at[tj], out_hbm.at[h, tj, pl.ds(nb0, nlb)]
                )

        def body(hh, carry):
            h0 = hh * 2
            h1 = h0 + 1
            fire(h1, 1, sem_b)
            drain(0, sem_a)
            transpose_chunk(0)
            write_out(h0)

            @pl.when(hh < hist // 2 - 1)
            def _():
                fire(h0 + 2, 0, sem_a)

            drain(1, sem_b)
            transpose_chunk(1)
            write_out(h1)
            return carry

        fire(0, 0, sem_a)
        lax.fori_loop(0, hist // 2, body, 0)

    return sc_gather


def _mask_body(x_ref, o_ref):
    o_ref[...] = x_ref[...] > 0


def kernel(x, table):
    bsz, hist = x.shape
    V, _ = table.shape
    x3 = x.reshape(NW, bsz // NW, hist)
    o6 = _make_sc_gather(bsz, hist, V)(x3, table)
    embs = (
        o6.transpose(2, 4, 0, 1, 3)
        .reshape(bsz, hist, D)
    )

    rows_blk = bsz // 16
    mask = pl.pallas_call(
        _mask_body,
        grid=(16,),
        in_specs=[pl.BlockSpec((rows_blk, hist), lambda i: (i, 0))],
        out_specs=pl.BlockSpec((rows_blk, hist), lambda i: (i, 0)),
        out_shape=jax.ShapeDtypeStruct((bsz, hist), jnp.bool_),
    )(x)
    return (embs, mask)


# speedup vs baseline: 2.4266x; 1.0151x over previous
import functools

import jax
import jax.numpy as jnp
from jax import lax
from jax.experimental import pallas as pl
from jax.experimental.pallas import tpu as pltpu
from jax.experimental.pallas import tpu_sc as plsc

D = 32
NW = 32
G = 128


def _make_sc_gather(bsz, hist, V):
    bw = bsz // NW
    nlb = bw // G
    nb_all = bsz // G
    mesh = plsc.VectorSubcoreMesh(core_axis_name="c", subcore_axis_name="s")

    @functools.partial(
        pl.kernel,
        mesh=mesh,
        out_type=jax.ShapeDtypeStruct((hist, 4, nb_all, 8, G), jnp.float32),
        scratch_types=[
            pltpu.VMEM((bw, hist), jnp.int32),
            pltpu.VMEM((hist, bw), jnp.int32),
            pltpu.VMEM((2, bw, D), jnp.float32),
            pltpu.VMEM((4, nlb, 8, G), jnp.float32),
            pltpu.SemaphoreType.DMA,
            pltpu.SemaphoreType.DMA,
        ],
        compiler_params=pltpu.CompilerParams(
            use_tc_tiling_on_sc=False, needs_layout_passes=False
        ),
    )
    def sc_gather(x_hbm, table_hbm, out_hbm, idx_raw, idx_t, gbuf, colbuf, sem_a, sem_b):
        wid = lax.axis_index("s") * 2 + lax.axis_index("c")
        pltpu.sync_copy(x_hbm.at[wid], idx_raw)
        nb0 = wid * nlb
        lane = lax.iota(jnp.int32, 16)

        def idx_tr_body(h, carry):
            cols = jnp.full((16,), h, jnp.int32)
            for q in range(bw // 16):
                v = plsc.load_gather(idx_raw, [lane + q * 16, cols])
                idx_t[h, pl.ds(q * 16, 16)] = v * 4
            return carry

        lax.fori_loop(0, hist, idx_tr_body, 0)

        def fire(h, buf, sem):
            for lb in range(nlb):
                pltpu.async_copy(
                    table_hbm.at[idx_t.at[h, pl.ds(lb * G, G)]],
                    gbuf.at[buf, pl.ds(lb * G, G)],
                    sem,
                )

        def drain(buf, sem):
            for lb in range(nlb):
                pltpu.make_async_copy(
                    table_hbm.at[pl.ds(0, G)],
                    gbuf.at[buf, pl.ds(lb * G, G)],
                    sem,
                ).wait()

        def transpose_chunk(buf):
            def emb_tr_body(d, carry):
                jm = (d + lane) & (D - 1)
                tj_v = jm >> 3
                dr_v = jm & 7
                for lb in range(nlb):
                    lb_v = jnp.full((16,), lb, jnp.int32)
                    for r in range(G // 16):
                        rows = lane + (lb * G + r * 16)
                        v = plsc.load_gather(gbuf.at[buf], [rows, jm])
                        plsc.store_scatter(
                            colbuf, [tj_v, lb_v, dr_v, lane + r * 16], v
                        )
                return carry

            lax.fori_loop(0, D, emb_tr_body, 0)

        def write_out(h):
            for tj in range(4):
                pltpu.sync_copy(
                    colbuf.at[tj], out_hbm.at[h, tj, pl.ds(nb0, nlb)]
                )

        def body(hh, carry):
            h0 = hh * 2
            h1 = h0 + 1
            fire(h1, 1, sem_b)
            drain(0, sem_a)
            transpose_chunk(0)
            write_out(h0)

            @pl.when(hh < hist // 2 - 1)
            def _():
                fire(h0 + 2, 0, sem_a)

            drain(1, sem_b)
            transpose_chunk(1)
            write_out(h1)
            return carry

        fire(0, 0, sem_a)
        lax.fori_loop(0, hist // 2, body, 0)

    return sc_gather


def _mask_body(x_ref, o_ref):
    o_ref[...] = x_ref[...] > 0


def kernel(x, table):
    bsz, hist = x.shape
    V, _ = table.shape
    x3 = x.reshape(NW, bsz // NW, hist)
    tab4 = jnp.pad(table, ((0, 0), (0, 3 * D))).reshape(4 * V, D)
    o6 = _make_sc_gather(bsz, hist, V)(x3, tab4)
    embs = (
        o6.transpose(2, 4, 0, 1, 3)
        .reshape(bsz, hist, D)
    )

    rows_blk = bsz // 16
    mask = pl.pallas_call(
        _mask_body,
        grid=(16,),
        in_specs=[pl.BlockSpec((rows_blk, hist), lambda i: (i, 0))],
        out_specs=pl.BlockSpec((rows_blk, hist), lambda i: (i, 0)),
        out_shape=jax.ShapeDtypeStruct((bsz, hist), jnp.bool_),
    )(x)
    return (embs, mask)
